# baseline (speedup 1.0000x reference)
import functools

import jax
import jax.numpy as jnp
from jax import lax
from jax.experimental import pallas as pl
from jax.experimental.pallas import tpu as pltpu
from jax.experimental.pallas import tpu_sc as plsc

VOCAB = 1000000
EMBED_DIM = 32
BATCH = 16384
HIST = 50

NW = 32
TOTAL = BATCH * HIST
PER_W = TOTAL // NW
CHUNK = 1600
NCH = PER_W // CHUNK

_mesh = plsc.VectorSubcoreMesh(core_axis_name="c", subcore_axis_name="s")


@functools.partial(
    pl.kernel,
    out_type=jax.ShapeDtypeStruct((TOTAL, EMBED_DIM), jnp.float32),
    mesh=_mesh,
    scratch_types=[
        pltpu.VMEM((CHUNK,), jnp.int32),
        pltpu.VMEM((CHUNK,), jnp.int32),
        pltpu.VMEM((CHUNK, EMBED_DIM), jnp.float32),
        pltpu.VMEM((CHUNK, EMBED_DIM), jnp.float32),
        pltpu.SemaphoreType.DMA,
        pltpu.SemaphoreType.DMA,
        pltpu.SemaphoreType.DMA,
        pltpu.SemaphoreType.DMA,
    ],
    compiler_params=pltpu.CompilerParams(use_tc_tiling_on_sc=False),
)
def _sc_gather(idx_hbm, tab_hbm, out_hbm, idx_a, idx_b, rows_a, rows_b,
               gsa, gsb, osa, osb):
    c = lax.axis_index("c")
    s = lax.axis_index("s")
    w = s * 2 + c
    base = w * PER_W

    idx = (idx_a, idx_b)
    rows = (rows_a, rows_b)
    gsem = (gsa, gsb)
    osem = (osa, osb)

    def off(j):
        return pl.multiple_of(base + j * CHUNK, 8)

    def idx_load(j, p):
        pltpu.sync_copy(idx_hbm.at[pl.ds(off(j), CHUNK)], idx[p])

    def gather(p):
        return pltpu.make_async_copy(tab_hbm.at[idx[p]], rows[p], gsem[p])

    def outw(j, p):
        return pltpu.make_async_copy(rows[p],
                                     out_hbm.at[pl.ds(off(j), CHUNK)],
                                     osem[p])

    idx_load(0, 0)
    gather(0).start()
    for j in range(NCH):
        p = j % 2
        n = 1 - p
        if j + 1 < NCH:
            idx_load(j + 1, n)
            if j >= 1:
                outw(j - 1, n).wait()
            gather(n).start()
        gather(p).wait()
        outw(j, p).start()
    outw(NCH - 2, (NCH - 2) % 2).wait()
    outw(NCH - 1, (NCH - 1) % 2).wait()


def kernel(PlayType, table):
    idx_flat = PlayType.reshape(TOTAL)
    out = _sc_gather(idx_flat, table)
    return out.reshape(BATCH, HIST, EMBED_DIM)


# speedup vs baseline: 1.3176x; 1.3176x over previous
import functools

import jax
import jax.numpy as jnp
from jax import lax
from jax.experimental import pallas as pl
from jax.experimental.pallas import tpu as pltpu
from jax.experimental.pallas import tpu_sc as plsc

VOCAB = 1000000
EMBED_DIM = 32
BATCH = 16384
HIST = 50

NW = 32
BB = 512
N_BC = BATCH // BB
N_CID = HIST * N_BC
PER_W = N_CID // NW

_mesh = plsc.VectorSubcoreMesh(core_axis_name="c", subcore_axis_name="s")

_i32 = jnp.int32


@functools.partial(
    pl.kernel,
    out_type=jax.ShapeDtypeStruct((HIST, EMBED_DIM, BATCH), jnp.float32),
    mesh=_mesh,
    scratch_types=[
        pltpu.VMEM((BB,), _i32),
        pltpu.VMEM((BB,), _i32),
        pltpu.VMEM((BB, EMBED_DIM), jnp.float32),
        pltpu.VMEM((BB, EMBED_DIM), jnp.float32),
        pltpu.VMEM((EMBED_DIM, BB), jnp.float32),
        pltpu.VMEM((EMBED_DIM, BB), jnp.float32),
        pltpu.SemaphoreType.DMA,
        pltpu.SemaphoreType.DMA,
        pltpu.SemaphoreType.DMA,
        pltpu.SemaphoreType.DMA,
    ],
    compiler_params=pltpu.CompilerParams(use_tc_tiling_on_sc=False,
                                         needs_layout_passes=False),
)
def _sc_embed(pt_t, tab_hbm, out_p, idx_a, idx_b, rows_a, rows_b,
              oblk_a, oblk_b, gsa, gsb, osa, osb):
    c = lax.axis_index("c")
    s = lax.axis_index("s")
    w = s * 2 + c

    idx = (idx_a, idx_b)
    rows = (rows_a, rows_b)
    oblk = (oblk_a, oblk_b)
    gsem = (gsa, gsb)
    osem = (osa, osb)

    def cid_of(j):
        return w * PER_W + j

    def idx_load(j, p):
        cid = cid_of(j)
        h = cid // N_BC
        b0 = pl.multiple_of((cid % N_BC) * BB, 128)
        pltpu.sync_copy(pt_t.at[h, pl.ds(b0, BB)], idx[p])

    def gather(p):
        return pltpu.make_async_copy(tab_hbm.at[idx[p]], rows[p], gsem[p])

    def outw(j, p):
        cid = cid_of(j)
        h = cid // N_BC
        b0 = pl.multiple_of((cid % N_BC) * BB, 128)
        return pltpu.make_async_copy(oblk[p],
                                     out_p.at[h, :, pl.ds(b0, BB)],
                                     osem[p])

    def transform(p):
        def body(bt, _):
            bvec = lax.iota(_i32, 16) + 16 * bt
            for d in range(EMBED_DIM):
                dvec = jnp.full((16,), d, _i32)
                vec = plsc.load_gather(rows[p], [bvec, dvec])
                oblk[p][d, pl.ds(16 * bt, 16)] = vec
            return 0

        lax.fori_loop(0, BB // 16, body, 0, unroll=False)

    n_pair = PER_W // 2

    idx_load(0, 0)
    gather(0).start()
    idx_load(1, 1)
    gather(1).start()

    def pair(t, _):
        j0 = 2 * t
        j1 = j0 + 1

        gather(0).wait()

        @pl.when(t > 0)
        def _():
            outw(j0 - 2, 0).wait()

        transform(0)
        outw(j0, 0).start()

        @pl.when(t + 1 < n_pair)
        def _():
            idx_load(j0 + 2, 0)
            gather(0).start()

        gather(1).wait()

        @pl.when(t > 0)
        def _():
            outw(j1 - 2, 1).wait()

        transform(1)
        outw(j1, 1).start()

        @pl.when(t + 1 < n_pair)
        def _():
            idx_load(j1 + 2, 1)
            gather(1).start()

        return 0

    lax.fori_loop(0, n_pair, pair, 0, unroll=False)
    outw(PER_W - 2, 0).wait()
    outw(PER_W - 1, 1).wait()


def kernel(PlayType, table):
    pt_t = PlayType.T
    out_p = _sc_embed(pt_t, table)
    return out_p.transpose(2, 0, 1)
